# baseline (speedup 1.0000x reference)
import functools

import jax
import jax.numpy as jnp
from jax import lax
from jax.experimental import pallas as pl
from jax.experimental.pallas import tpu as pltpu
from jax.experimental.pallas import tpu_sc as plsc

_INFO = plsc.get_sparse_core_info()
_NC = _INFO.num_cores
_NS = _INFO.num_subcores
_NW = _NC * _NS
_L = 16


def _make_gather(batch: int, hist: int, dim: int):
    bblk = batch // _NW
    assert batch == _NW * bblk and bblk == 128 and dim == 64
    assert hist % 2 == 0
    mesh = plsc.VectorSubcoreMesh(core_axis_name="c", subcore_axis_name="s")

    @functools.partial(
        pl.kernel,
        mesh=mesh,
        out_type=jax.ShapeDtypeStruct(
            (hist, dim // 8, _NW, 8, bblk), jnp.float32),
        scratch_types=[
            pltpu.VMEM((bblk, hist), jnp.int32),
            pltpu.VMEM((hist, bblk), jnp.int32),
            [pltpu.VMEM((bblk, dim), jnp.float32) for _ in range(2)],
            [pltpu.VMEM((dim // 8, 8, bblk), jnp.float32) for _ in range(2)],
            [pltpu.SemaphoreType.DMA for _ in range(2)],
            [pltpu.SemaphoreType.DMA for _ in range(2)],
        ],
        compiler_params=pltpu.CompilerParams(
            use_tc_tiling_on_sc=False, needs_layout_passes=False),
    )
    def gather_kernel(tok_hbm, table_hbm, out_hbm, idx_raw, idx_t,
                      rows, dmaj, sg, ss):
        w = lax.axis_index("s") * _NC + lax.axis_index("c")
        base16 = lax.iota(jnp.int32, _L)
        row_ids = [base16 + k * _L for k in range(bblk // _L)]

        pltpu.sync_copy(tok_hbm.at[pl.ds(w * bblk, bblk)], idx_raw)

        @pl.loop(0, hist)
        def _(hh):
            col = jnp.full((_L,), 0, jnp.int32) + hh
            for k in range(bblk // _L):
                v = plsc.load_gather(idx_raw, [row_ids[k], col])
                idx_t[hh, pl.ds(k * _L, _L)] = v

        def start_gather(h, b):
            pltpu.async_copy(table_hbm.at[idx_t.at[h]], rows[b], sg[b])

        def transpose(b):
            for d in range(dim):
                col = jnp.full((_L,), d, jnp.int32)
                for k in range(bblk // _L):
                    v = plsc.load_gather(rows[b], [row_ids[k], col])
                    dmaj[b][d // 8, d % 8, pl.ds(k * _L, _L)] = v

        start_gather(0, 0)

        @pl.loop(0, hist // 2)
        def _(g):
            for b in range(2):
                h = g * 2 + b
                nb = 1 - b

                @pl.when(h + 1 < hist)
                def _():
                    start_gather(h + 1, nb)

                pltpu.make_async_copy(
                    table_hbm.at[idx_t.at[h]], rows[b], sg[b]).wait()

                @pl.when(g > 0)
                def _():
                    pltpu.make_async_copy(
                        dmaj[b], out_hbm.at[0, :, w], ss[b]).wait()

                transpose(b)
                pltpu.async_copy(dmaj[b], out_hbm.at[h, :, w], ss[b])

        for b in range(2):
            pltpu.make_async_copy(dmaj[b], out_hbm.at[0, :, w], ss[b]).wait()

    return gather_kernel


def kernel(token_ids, weight):
    batch, hist = token_ids.shape
    _, dim = weight.shape
    out5 = _make_gather(batch, hist, dim)(token_ids, weight)
    return out5.transpose(2, 4, 0, 1, 3).reshape(batch, hist, dim)


# speedup vs baseline: 1.5350x; 1.5350x over previous
import functools

import jax
import jax.numpy as jnp
from jax import lax
from jax.experimental import pallas as pl
from jax.experimental.pallas import tpu as pltpu
from jax.experimental.pallas import tpu_sc as plsc

_INFO = plsc.get_sparse_core_info()
_NC = _INFO.num_cores
_NS = _INFO.num_subcores
_NW = _NC * _NS
_L = 16


def _make_gather(batch: int, hist: int, dim: int):
    bblk = batch // _NW
    assert batch == _NW * bblk and bblk == 128 and dim == 64
    assert hist % 2 == 0
    mesh = plsc.VectorSubcoreMesh(core_axis_name="c", subcore_axis_name="s")

    @functools.partial(
        pl.kernel,
        mesh=mesh,
        out_type=jax.ShapeDtypeStruct(
            (hist, dim // 8, _NW, 8, bblk), jnp.float32),
        scratch_types=[
            pltpu.VMEM((bblk, hist), jnp.int32),
            pltpu.VMEM((hist, bblk), jnp.int32),
            [pltpu.VMEM((bblk, dim), jnp.float32) for _ in range(2)],
            [pltpu.VMEM((dim // 8, 8, bblk), jnp.float32) for _ in range(2)],
            [pltpu.SemaphoreType.DMA for _ in range(2)],
            [pltpu.SemaphoreType.DMA for _ in range(2)],
        ],
        compiler_params=pltpu.CompilerParams(
            use_tc_tiling_on_sc=False, needs_layout_passes=False),
    )
    def gather_kernel(tok_hbm, table_hbm, out_hbm, idx_raw, idx_t,
                      rows, dmaj, sg, ss):
        w = lax.axis_index("s") * _NC + lax.axis_index("c")
        base16 = lax.iota(jnp.int32, _L)
        row_ids = [base16 + k * _L for k in range(bblk // _L)]

        pltpu.sync_copy(tok_hbm.at[pl.ds(w * bblk, bblk)], idx_raw)

        @pl.loop(0, hist)
        def _(hh):
            col = jnp.full((_L,), 0, jnp.int32) + hh
            for k in range(bblk // _L):
                v = plsc.load_gather(idx_raw, [row_ids[k], col])
                idx_t[hh, pl.ds(k * _L, _L)] = v

        def start_gather(h, b):
            pltpu.async_copy(table_hbm.at[idx_t.at[h]], rows[b], sg[b])

        rb_ids = [(base16 + k * _L) >> 3 for k in range(dim // _L)]
        ri_ids = [(base16 + k * _L) & 7 for k in range(dim // _L)]

        def transpose(b):
            @plsc.parallel_loop(0, bblk, unroll=2)
            def _(ci):
                civ = jnp.full((_L,), 0, jnp.int32) + ci
                for k in range(dim // _L):
                    v = rows[b][ci, pl.ds(k * _L, _L)]
                    plsc.store_scatter(dmaj[b], [rb_ids[k], ri_ids[k], civ], v)

        start_gather(0, 0)

        @pl.loop(0, hist // 2)
        def _(g):
            for b in range(2):
                h = g * 2 + b
                nb = 1 - b

                @pl.when(h + 1 < hist)
                def _():
                    start_gather(h + 1, nb)

                pltpu.make_async_copy(
                    table_hbm.at[idx_t.at[h]], rows[b], sg[b]).wait()

                @pl.when(g > 0)
                def _():
                    pltpu.make_async_copy(
                        dmaj[b], out_hbm.at[0, :, w], ss[b]).wait()

                transpose(b)
                pltpu.async_copy(dmaj[b], out_hbm.at[h, :, w], ss[b])

        for b in range(2):
            pltpu.make_async_copy(dmaj[b], out_hbm.at[0, :, w], ss[b]).wait()

    return gather_kernel


def kernel(token_ids, weight):
    batch, hist = token_ids.shape
    _, dim = weight.shape
    out5 = _make_gather(batch, hist, dim)(token_ids, weight)
    return out5.transpose(2, 4, 0, 1, 3).reshape(batch, hist, dim)


# speedup vs baseline: 2.0322x; 1.3239x over previous
import functools

import jax
import jax.numpy as jnp
from jax import lax
from jax.experimental import pallas as pl
from jax.experimental.pallas import tpu as pltpu
from jax.experimental.pallas import tpu_sc as plsc

_INFO = plsc.get_sparse_core_info()
_NC = _INFO.num_cores
_NS = _INFO.num_subcores
_NW = _NC * _NS
_L = 16


def _make_gather(batch: int, hist: int, dim: int):
    bblk = batch // _NW
    assert batch == _NW * bblk and bblk == 128 and dim == 64
    assert hist % 2 == 0
    mesh = plsc.VectorSubcoreMesh(core_axis_name="c", subcore_axis_name="s")

    @functools.partial(
        pl.kernel,
        mesh=mesh,
        out_type=jax.ShapeDtypeStruct(
            (hist, dim // 8, _NW, 8, bblk), jnp.float32),
        scratch_types=[
            pltpu.VMEM((bblk, hist), jnp.int32),
            pltpu.VMEM((hist, bblk), jnp.int32),
            [pltpu.VMEM((bblk, dim), jnp.float32) for _ in range(2)],
            [pltpu.VMEM((dim // 8, 8, bblk), jnp.float32) for _ in range(2)],
            [pltpu.SemaphoreType.DMA for _ in range(2)],
            [pltpu.SemaphoreType.DMA for _ in range(2)],
        ],
        compiler_params=pltpu.CompilerParams(
            use_tc_tiling_on_sc=False, needs_layout_passes=False),
    )
    def gather_kernel(tok_hbm, table_hbm, out_hbm, idx_raw, idx_t,
                      rows, dmaj, sg, ss):
        w = lax.axis_index("s") * _NC + lax.axis_index("c")
        base16 = lax.iota(jnp.int32, _L)
        row_ids = [base16 + k * _L for k in range(bblk // _L)]

        pltpu.sync_copy(tok_hbm.at[pl.ds(w * bblk, bblk)], idx_raw)

        @pl.loop(0, hist)
        def _(hh):
            col = jnp.full((_L,), 0, jnp.int32) + hh
            for k in range(bblk // _L):
                v = plsc.load_gather(idx_raw, [row_ids[k], col])
                idx_t[hh, pl.ds(k * _L, _L)] = v

        def start_gather(h, b):
            pltpu.async_copy(table_hbm.at[idx_t.at[h]], rows[b], sg[b])

        skew = [(base16 + s) & (_L - 1) for s in range(_L)]

        def transpose(b):
            @plsc.parallel_loop(0, bblk // _L)
            def _(cb):
                rowv = base16 + cb * _L
                for d0 in range(0, dim, _L):
                    for s in range(_L):
                        dv = skew[s] + d0
                        v = plsc.load_gather(rows[b], [rowv, dv])
                        plsc.store_scatter(
                            dmaj[b], [dv >> 3, dv & 7, rowv], v)

        start_gather(0, 0)

        @pl.loop(0, hist // 2)
        def _(g):
            for b in range(2):
                h = g * 2 + b
                nb = 1 - b

                @pl.when(h + 1 < hist)
                def _():
                    start_gather(h + 1, nb)

                pltpu.make_async_copy(
                    table_hbm.at[idx_t.at[h]], rows[b], sg[b]).wait()

                @pl.when(g > 0)
                def _():
                    pltpu.make_async_copy(
                        dmaj[b], out_hbm.at[0, :, w], ss[b]).wait()

                transpose(b)
                pltpu.async_copy(dmaj[b], out_hbm.at[h, :, w], ss[b])

        for b in range(2):
            pltpu.make_async_copy(dmaj[b], out_hbm.at[0, :, w], ss[b]).wait()

    return gather_kernel


def kernel(token_ids, weight):
    batch, hist = token_ids.shape
    _, dim = weight.shape
    out5 = _make_gather(batch, hist, dim)(token_ids, weight)
    return out5.transpose(2, 4, 0, 1, 3).reshape(batch, hist, dim)
